# baseline (speedup 1.0000x reference)
import functools

import jax
import jax.numpy as jnp
from jax import lax
from jax.experimental import pallas as pl
from jax.experimental.pallas import tpu as pltpu
from jax.experimental.pallas import tpu_sc as plsc

WORD_DIM = 128
CHAR_DIM = 16
CHAR_VOCAB = 256
MAX_WORD_LEN = 16
CONV_WIDTHS = (2, 3, 4)
CONV_OUT = 64
POS_SLOTS = 16
CONV_COLS = len(CONV_WIDTHS) * POS_SLOTS * CONV_OUT


def _sc_gather(widx, cidx, word_table, char_table):
    N = widx.shape[0]
    info = plsc.get_sparse_core_info()
    NW = info.num_cores * info.num_subcores
    n_per = N // NW
    WCH = 80 if n_per % 160 == 0 else 40
    CW = 200 if n_per % 200 == 0 else 100
    n_w_iters = n_per // WCH
    n_c_chunks = n_per // CW
    row = CHAR_DIM * MAX_WORD_LEN
    mesh = plsc.VectorSubcoreMesh(core_axis_name="c", subcore_axis_name="s")

    @functools.partial(
        pl.kernel, mesh=mesh,
        out_type=[
            jax.ShapeDtypeStruct((N, WORD_DIM), jnp.float32),
            jax.ShapeDtypeStruct((2 * N * WORD_DIM,), jnp.float32),
        ],
        scratch_types=[
            pltpu.VMEM((n_per,), jnp.int32),
            pltpu.VMEM((WCH, WORD_DIM), jnp.float32),
            pltpu.VMEM((WCH, WORD_DIM), jnp.float32),
            pltpu.VMEM((CHAR_VOCAB * CHAR_DIM,), jnp.float32),
            pltpu.VMEM((n_per * MAX_WORD_LEN,), jnp.int32),
            pltpu.VMEM((CW * WORD_DIM,), jnp.float32),
            pltpu.VMEM((CW * WORD_DIM,), jnp.float32),
            pltpu.SemaphoreType.DMA,
            pltpu.SemaphoreType.DMA,
        ],
        compiler_params=pltpu.CompilerParams(needs_layout_passes=False),
    )
    def k(widx_hbm, cidx_hbm, wtab_hbm, ctab_hbm, wout_hbm, cout_hbm,
          widx_v, wrows0_v, wrows1_v, ctab_v, cidx_v, clo_v, chi_v,
          sem0, sem1):
        wid = lax.axis_index("s") * info.num_cores + lax.axis_index("c")
        wbase = wid * n_per

        pltpu.sync_copy(ctab_hbm, ctab_v)
        pltpu.sync_copy(widx_hbm.at[pl.ds(wbase, n_per)], widx_v)
        pltpu.sync_copy(
            cidx_hbm.at[pl.ds(wbase * MAX_WORD_LEN, n_per * MAX_WORD_LEN)],
            cidx_v)

        def _gather(j, rows_v, sem):
            pltpu.async_copy(
                wtab_hbm.at[widx_v.at[pl.ds(j * WCH, WCH)]], rows_v, sem)

        def _gwait(rows_v, sem):
            pltpu.make_async_copy(
                wtab_hbm.at[pl.ds(0, WCH)], rows_v, sem).wait()

        _gather(0, wrows0_v, sem0)
        n_pairs = n_w_iters // 2

        def wbody(p, carry):
            j0 = p * 2
            _gather(j0 + 1, wrows1_v, sem1)
            _gwait(wrows0_v, sem0)
            pltpu.sync_copy(wrows0_v,
                            wout_hbm.at[pl.ds(wbase + j0 * WCH, WCH)])

            @pl.when(p < n_pairs - 1)
            def _():
                _gather(j0 + 2, wrows0_v, sem0)

            _gwait(wrows1_v, sem1)
            pltpu.sync_copy(wrows1_v,
                            wout_hbm.at[pl.ds(wbase + (j0 + 1) * WCH, WCH)])
            return carry

        lax.fori_loop(0, n_pairs, wbody, 0)

        def cchunk(ch, carry):
            @plsc.parallel_loop(0, CW, 1, unroll=2)
            def cword(w):
                gw = ch * CW + w
                flat = cidx_v[pl.ds(gw * MAX_WORD_LEN, MAX_WORD_LEN)]
                for d in range(CHAR_DIM):
                    vals = plsc.load_gather(ctab_v, [flat + d])
                    if d < CHAR_DIM // 2:
                        clo_v[pl.ds(w * WORD_DIM + d * MAX_WORD_LEN,
                                    MAX_WORD_LEN)] = vals
                    else:
                        chi_v[pl.ds(w * WORD_DIM + (d - CHAR_DIM // 2) *
                                    MAX_WORD_LEN, MAX_WORD_LEN)] = vals
            base = (wbase + ch * CW) * WORD_DIM
            pltpu.sync_copy(clo_v, cout_hbm.at[pl.ds(base, CW * WORD_DIM)])
            pltpu.sync_copy(
                chi_v, cout_hbm.at[pl.ds(N * WORD_DIM + base, CW * WORD_DIM)])
            return carry

        lax.fori_loop(0, n_c_chunks, cchunk, 0)

    return k(widx, cidx, word_table, char_table.reshape(-1))


def _build_conv_matrix(k2, k3, k4):
    rows = CHAR_DIM * MAX_WORD_LEN
    Bm = jnp.zeros((rows, CONV_COLS), jnp.float32)
    for ki, W in enumerate((k2, k3, k4)):
        kw = W.shape[2]
        Wr = jnp.transpose(W, (2, 1, 0)).reshape(kw * CHAR_DIM, CONV_OUT)
        for t in range(MAX_WORD_LEN - kw + 1):
            Bm = lax.dynamic_update_slice(
                Bm, Wr, (t * CHAR_DIM, (ki * POS_SLOTS + t) * CONV_OUT))
    Bm = Bm.reshape(MAX_WORD_LEN, CHAR_DIM, CONV_COLS)
    Bm = jnp.transpose(Bm, (1, 0, 2)).reshape(rows, CONV_COLS)
    return Bm


def _tc_conv(w_flat, c2, Bmat, Bsz, S, blk_off=0, out_prev=None):
    N = w_flat.shape[0]
    RB = 16
    TN = RB * S
    OUT = WORD_DIM + len(CONV_WIDTHS) * CONV_OUT
    nblk = N // TN

    def body(w_ref, clo_ref, chi_ref, b_ref, o_ref):
        x1 = clo_ref[:].astype(jnp.bfloat16)
        x2 = chi_ref[:].astype(jnp.bfloat16)
        for ki in range(3):
            acc = None
            for j in range(4):
                lo = (ki * 4 + j) * 256
                y = (jnp.dot(x1, b_ref[0:WORD_DIM, lo:lo + 256],
                             preferred_element_type=jnp.float32) +
                     jnp.dot(x2, b_ref[WORD_DIM:2 * WORD_DIM, lo:lo + 256],
                             preferred_element_type=jnp.float32))
                acc = y if acc is None else jnp.maximum(acc, y)
            m = jnp.maximum(acc[:, 0:128], acc[:, 128:256])
            m = jnp.maximum(m[:, 0:64], m[:, 64:128])
            m = jnp.maximum(m, 0.0)
            c0 = WORD_DIM + ki * CONV_OUT
            o_ref[:, :, c0:c0 + CONV_OUT] = m.reshape(RB, S, CONV_OUT)
        o_ref[:, :, 0:WORD_DIM] = w_ref[:].reshape(RB, S, WORD_DIM)

    in_specs = [
        pl.BlockSpec((TN, WORD_DIM), lambda i: (i, 0)),
        pl.BlockSpec((TN, WORD_DIM), lambda i: (i, 0)),
        pl.BlockSpec((TN, WORD_DIM), lambda i, _n=nblk: (_n + i, 0)),
        pl.BlockSpec((2 * WORD_DIM, CONV_COLS), lambda i: (0, 0)),
    ]
    args = [w_flat, c2, c2, Bmat]
    kwargs = {}
    run_body = body
    if out_prev is not None:
        in_specs = [pl.BlockSpec((8, S, OUT), lambda i: (0, 0, 0))] + in_specs
        args = [out_prev] + args
        kwargs["input_output_aliases"] = {0: 0}

        def run_body(prev_ref, *refs):
            body(*refs)

    return pl.pallas_call(
        run_body,
        grid=(nblk,),
        in_specs=in_specs,
        out_specs=pl.BlockSpec((RB, S, OUT),
                               lambda i, _o=blk_off: (_o + i, 0, 0)),
        out_shape=jax.ShapeDtypeStruct((Bsz, S, OUT), jnp.float32),
        compiler_params=pltpu.CompilerParams(
            dimension_semantics=("arbitrary",),
        ),
        **kwargs,
    )(*args)


def kernel(word_input, character_input, word_table, char_table,
           kernel_2, kernel_3, kernel_4):
    Bsz, S = word_input.shape
    N = Bsz * S
    widx = word_input.reshape(N).astype(jnp.int32)
    cidx = (character_input.astype(jnp.int32) * CHAR_DIM).reshape(
        N * MAX_WORD_LEN)
    Bmat = _build_conv_matrix(kernel_2, kernel_3, kernel_4).astype(jnp.bfloat16)
    NSPLIT = 4
    h = N // NSPLIT
    hb = Bsz // NSPLIT
    out = None
    for p in range(NSPLIT):
        widx_h = lax.dynamic_slice_in_dim(widx, p * h, h)
        cidx_h = lax.dynamic_slice_in_dim(cidx, p * h * MAX_WORD_LEN,
                                          h * MAX_WORD_LEN)
        w_flat, c_rows = _sc_gather(widx_h, cidx_h, word_table, char_table)
        c2 = c_rows.reshape(2 * h, WORD_DIM)
        out = _tc_conv(w_flat, c2, Bmat, Bsz, S,
                       blk_off=p * (hb // 16), out_prev=out)
    return out


# speedup vs baseline: 1.0034x; 1.0034x over previous
import functools

import jax
import jax.numpy as jnp
from jax import lax
from jax.experimental import pallas as pl
from jax.experimental.pallas import tpu as pltpu
from jax.experimental.pallas import tpu_sc as plsc

WORD_DIM = 128
CHAR_DIM = 16
CHAR_VOCAB = 256
MAX_WORD_LEN = 16
CONV_WIDTHS = (2, 3, 4)
CONV_OUT = 64
POS_SLOTS = 16
CONV_COLS = len(CONV_WIDTHS) * POS_SLOTS * CONV_OUT


def _sc_gather(widx, cidx, word_table, char_table):
    N = widx.shape[0]
    info = plsc.get_sparse_core_info()
    NW = info.num_cores * info.num_subcores
    n_per = N // NW
    WCH = 80
    CW = 200
    n_w_iters = n_per // WCH
    n_c_chunks = n_per // CW
    row = CHAR_DIM * MAX_WORD_LEN
    mesh = plsc.VectorSubcoreMesh(core_axis_name="c", subcore_axis_name="s")

    @functools.partial(
        pl.kernel, mesh=mesh,
        out_type=[
            jax.ShapeDtypeStruct((N, WORD_DIM), jnp.float32),
            jax.ShapeDtypeStruct((2 * N * WORD_DIM,), jnp.float32),
        ],
        scratch_types=[
            pltpu.VMEM((n_per,), jnp.int32),
            pltpu.VMEM((WCH, WORD_DIM), jnp.float32),
            pltpu.VMEM((WCH, WORD_DIM), jnp.float32),
            pltpu.VMEM((CHAR_VOCAB * CHAR_DIM,), jnp.float32),
            pltpu.VMEM((n_per * MAX_WORD_LEN,), jnp.int32),
            pltpu.VMEM((CW * WORD_DIM,), jnp.float32),
            pltpu.VMEM((CW * WORD_DIM,), jnp.float32),
            pltpu.SemaphoreType.DMA,
            pltpu.SemaphoreType.DMA,
        ],
        compiler_params=pltpu.CompilerParams(needs_layout_passes=False),
    )
    def k(widx_hbm, cidx_hbm, wtab_hbm, ctab_hbm, wout_hbm, cout_hbm,
          widx_v, wrows0_v, wrows1_v, ctab_v, cidx_v, clo_v, chi_v,
          sem0, sem1):
        wid = lax.axis_index("s") * info.num_cores + lax.axis_index("c")
        wbase = wid * n_per

        pltpu.sync_copy(ctab_hbm, ctab_v)
        pltpu.sync_copy(widx_hbm.at[pl.ds(wbase, n_per)], widx_v)
        pltpu.sync_copy(
            cidx_hbm.at[pl.ds(wbase * MAX_WORD_LEN, n_per * MAX_WORD_LEN)],
            cidx_v)

        def _gather(j, rows_v, sem):
            pltpu.async_copy(
                wtab_hbm.at[widx_v.at[pl.ds(j * WCH, WCH)]], rows_v, sem)

        def _gwait(rows_v, sem):
            pltpu.make_async_copy(
                wtab_hbm.at[pl.ds(0, WCH)], rows_v, sem).wait()

        _gather(0, wrows0_v, sem0)
        n_pairs = n_w_iters // 2

        def wbody(p, carry):
            j0 = p * 2
            _gather(j0 + 1, wrows1_v, sem1)
            _gwait(wrows0_v, sem0)
            pltpu.sync_copy(wrows0_v,
                            wout_hbm.at[pl.ds(wbase + j0 * WCH, WCH)])

            @pl.when(p < n_pairs - 1)
            def _():
                _gather(j0 + 2, wrows0_v, sem0)

            _gwait(wrows1_v, sem1)
            pltpu.sync_copy(wrows1_v,
                            wout_hbm.at[pl.ds(wbase + (j0 + 1) * WCH, WCH)])
            return carry

        lax.fori_loop(0, n_pairs, wbody, 0)

        def cchunk(ch, carry):
            @plsc.parallel_loop(0, CW, 1, unroll=2)
            def cword(w):
                gw = ch * CW + w
                flat = cidx_v[pl.ds(gw * MAX_WORD_LEN, MAX_WORD_LEN)]
                for d in range(CHAR_DIM):
                    vals = plsc.load_gather(ctab_v, [flat + d])
                    if d < CHAR_DIM // 2:
                        clo_v[pl.ds(w * WORD_DIM + d * MAX_WORD_LEN,
                                    MAX_WORD_LEN)] = vals
                    else:
                        chi_v[pl.ds(w * WORD_DIM + (d - CHAR_DIM // 2) *
                                    MAX_WORD_LEN, MAX_WORD_LEN)] = vals
            base = (wbase + ch * CW) * WORD_DIM
            pltpu.sync_copy(clo_v, cout_hbm.at[pl.ds(base, CW * WORD_DIM)])
            pltpu.sync_copy(
                chi_v, cout_hbm.at[pl.ds(N * WORD_DIM + base, CW * WORD_DIM)])
            return carry

        lax.fori_loop(0, n_c_chunks, cchunk, 0)

    return k(widx, cidx, word_table, char_table.reshape(-1))


def _build_conv_matrix(k2, k3, k4):
    rows = CHAR_DIM * MAX_WORD_LEN
    Bm = jnp.zeros((rows, CONV_COLS), jnp.float32)
    for ki, W in enumerate((k2, k3, k4)):
        kw = W.shape[2]
        Wr = jnp.transpose(W, (2, 1, 0)).reshape(kw * CHAR_DIM, CONV_OUT)
        for t in range(MAX_WORD_LEN - kw + 1):
            Bm = lax.dynamic_update_slice(
                Bm, Wr, (t * CHAR_DIM, (ki * POS_SLOTS + t) * CONV_OUT))
    Bm = Bm.reshape(MAX_WORD_LEN, CHAR_DIM, CONV_COLS)
    Bm = jnp.transpose(Bm, (1, 0, 2)).reshape(rows, CONV_COLS)
    return Bm


def _tc_conv(w_flat, c2, Bmat, Bsz, S, blk_off=0, out_prev=None):
    N = w_flat.shape[0]
    RB = 16
    TN = RB * S
    OUT = WORD_DIM + len(CONV_WIDTHS) * CONV_OUT
    nblk = N // TN

    def body(w_ref, clo_ref, chi_ref, b_ref, o_ref):
        x1 = clo_ref[:].astype(jnp.bfloat16)
        x2 = chi_ref[:].astype(jnp.bfloat16)
        for ki in range(3):
            acc = None
            for j in range(4):
                lo = (ki * 4 + j) * 256
                y = (jnp.dot(x1, b_ref[0:WORD_DIM, lo:lo + 256],
                             preferred_element_type=jnp.float32) +
                     jnp.dot(x2, b_ref[WORD_DIM:2 * WORD_DIM, lo:lo + 256],
                             preferred_element_type=jnp.float32))
                acc = y if acc is None else jnp.maximum(acc, y)
            m = jnp.maximum(acc[:, 0:128], acc[:, 128:256])
            m = jnp.maximum(m[:, 0:64], m[:, 64:128])
            m = jnp.maximum(m, 0.0)
            c0 = WORD_DIM + ki * CONV_OUT
            o_ref[:, :, c0:c0 + CONV_OUT] = m.reshape(RB, S, CONV_OUT)
        o_ref[:, :, 0:WORD_DIM] = w_ref[:].reshape(RB, S, WORD_DIM)

    in_specs = [
        pl.BlockSpec((TN, WORD_DIM), lambda i: (i, 0)),
        pl.BlockSpec((TN, WORD_DIM), lambda i: (i, 0)),
        pl.BlockSpec((TN, WORD_DIM), lambda i, _n=nblk: (_n + i, 0)),
        pl.BlockSpec((2 * WORD_DIM, CONV_COLS), lambda i: (0, 0)),
    ]
    args = [w_flat, c2, c2, Bmat]
    kwargs = {}
    run_body = body
    if out_prev is not None:
        in_specs = [pl.BlockSpec((8, S, OUT), lambda i: (0, 0, 0))] + in_specs
        args = [out_prev] + args
        kwargs["input_output_aliases"] = {0: 0}

        def run_body(prev_ref, *refs):
            body(*refs)

    return pl.pallas_call(
        run_body,
        grid=(nblk,),
        in_specs=in_specs,
        out_specs=pl.BlockSpec((RB, S, OUT),
                               lambda i, _o=blk_off: (_o + i, 0, 0)),
        out_shape=jax.ShapeDtypeStruct((Bsz, S, OUT), jnp.float32),
        compiler_params=pltpu.CompilerParams(
            dimension_semantics=("arbitrary",),
        ),
        **kwargs,
    )(*args)


def kernel(word_input, character_input, word_table, char_table,
           kernel_2, kernel_3, kernel_4):
    Bsz, S = word_input.shape
    N = Bsz * S
    widx = word_input.reshape(N).astype(jnp.int32)
    cidx = (character_input.astype(jnp.int32) * CHAR_DIM).reshape(
        N * MAX_WORD_LEN)
    Bmat = _build_conv_matrix(kernel_2, kernel_3, kernel_4).astype(jnp.bfloat16)
    h = N // 2
    hb = Bsz // 2
    out = None
    for p in range(2):
        widx_h = lax.dynamic_slice_in_dim(widx, p * h, h)
        cidx_h = lax.dynamic_slice_in_dim(cidx, p * h * MAX_WORD_LEN,
                                          h * MAX_WORD_LEN)
        w_flat, c_rows = _sc_gather(widx_h, cidx_h, word_table, char_table)
        c2 = c_rows.reshape(2 * h, WORD_DIM)
        out = _tc_conv(w_flat, c2, Bmat, Bsz, S,
                       blk_off=p * (hb // 16), out_prev=out)
    return out
